# baseline (speedup 1.0000x reference)
import jax
import jax.numpy as jnp
from jax import lax
from jax.experimental import pallas as pl
from jax.experimental.pallas import tpu as pltpu
from jax.experimental.pallas import tpu_sc as plsc

N0 = 10000
N1 = 5000
NF = 128
NH = 128
NCLS = 40
MAXW = 64

SC_CORES = 2
SC_SUBCORES = 16
NWORK = SC_CORES * SC_SUBCORES
CK = 128


def _cdiv(a, b):
    return (a + b - 1) // b


def _make_sc_spmm(d, ch, n_out, weighted, with_deg):
    n_acc = _cdiv(n_out + 1, SC_SUBCORES * 8) * SC_SUBCORES * 8
    rps = n_acc // SC_SUBCORES
    mesh = plsc.VectorSubcoreMesh(core_axis_name="c", subcore_axis_name="s")

    dr = n_acc // CK
    out_type = [jax.ShapeDtypeStruct((SC_CORES, n_acc, d), jnp.float32)]
    if with_deg:
        out_type.append(jax.ShapeDtypeStruct((NWORK, dr, CK), jnp.float32))

    scratch = [
        pltpu.VMEM((ch, CK), jnp.int32),
        pltpu.VMEM((ch, CK), jnp.int32),
    ]
    if weighted:
        scratch.append(pltpu.VMEM((ch, CK), jnp.float32))
    scratch.append(pltpu.VMEM((2, CK, d), jnp.float32))
    if with_deg:
        scratch.append(pltpu.VMEM((dr, CK), jnp.float32))
    scratch.append(pltpu.VMEM_SHARED((n_acc, d), jnp.float32))
    scratch.append(pltpu.SemaphoreType.DMA)
    scratch.append(pltpu.SemaphoreType.DMA)
    scratch.append(pltpu.SemaphoreType.DMA)
    scratch.append(pltpu.SemaphoreType.DMA)

    def body(*refs):
        it = iter(refs)
        y_h = next(it)
        src_h = next(it)
        dst_h = next(it)
        w_h = next(it) if weighted else None
        acc_h = next(it)
        deg_h = next(it) if with_deg else None
        src_v = next(it)
        dst_v = next(it)
        w_v = next(it) if weighted else None
        rows2_v = next(it)
        degtile = next(it) if with_deg else None
        acc = next(it)
        sems = (next(it), next(it))
        ssems = (next(it), next(it))
        bufs = (rows2_v.at[0], rows2_v.at[1])

        c = lax.axis_index("c")
        s = lax.axis_index("s")
        wid = s * SC_CORES + c

        def zrow(e, carry):
            z = jnp.zeros((16,), jnp.float32)
            for j in range(d // 16):
                rows2_v[0, e, pl.ds(j * 16, 16)] = z
            return carry

        lax.fori_loop(0, CK, zrow, 0)
        if with_deg:
            def zdeg(r, carry):
                z = jnp.zeros((16,), jnp.float32)
                for j in range(CK // 16):
                    degtile[r, pl.ds(j * 16, 16)] = z
                return carry

            lax.fori_loop(0, dr, zdeg, 0)
        base = s * rps
        off = 0
        while off < rps:
            csz = min(rps - off, CK)
            pltpu.sync_copy(bufs[0].at[pl.ds(0, csz)],
                            acc.at[pl.ds(base + off, csz)])
            off += csz
        plsc.subcore_barrier()

        pltpu.sync_copy(src_h.at[pl.ds(wid * ch, ch)], src_v)
        pltpu.sync_copy(dst_h.at[pl.ds(wid * ch, ch)], dst_v)
        if weighted:
            pltpu.sync_copy(w_h.at[pl.ds(wid * ch, ch)], w_v)

        def issue(jc, b):
            return pltpu.async_copy(y_h.at[src_v.at[jc]], bufs[b], sems[b])

        def wait(jc, b):
            pltpu.make_async_copy(y_h.at[src_v.at[jc]], bufs[b],
                                  sems[b]).wait()

        def process(jc, b):
            buf = bufs[b]
            wait(jc, b)
            if weighted:
                def scale(g, cc):
                    wv = w_v[jc, pl.ds(g * 16, 16)]
                    dv = dst_v[jc, pl.ds(g * 16, 16)] if with_deg else None
                    for e16 in range(16):
                        we = wv[e16]
                        e = g * 16 + e16
                        for j in range(d // 16):
                            sl = pl.ds(j * 16, 16)
                            buf[e, sl] = buf[e, sl] * we
                        if with_deg:
                            de = dv[e16]
                            r = de >> 7
                            cb = ((de >> 4) & 7) * 16
                            lane = de & 15
                            lanes = lax.iota(jnp.int32, 16)
                            cur = degtile[r, pl.ds(cb, 16)]
                            degtile[r, pl.ds(cb, 16)] = cur + jnp.where(
                                lanes == lane, we, 0.0)
                    return cc

                lax.fori_loop(0, CK // 16, scale, 0)

        def sc_start(jc, b):
            pltpu.async_copy(bufs[b], acc.at[dst_v.at[jc]], ssems[b],
                             add=True)

        def sc_wait(jc, b):
            pltpu.make_async_copy(bufs[b], acc.at[dst_v.at[jc]],
                                  ssems[b]).wait()

        def step(j, b):
            process(j, b)
            sc_start(j, b)

            @pl.when(j > 0)
            def _():
                sc_wait(j - 1, 1 - b)

            @pl.when(j + 1 < ch)
            def _():
                issue(j + 1, 1 - b)

        assert ch % 2 == 0 and ch >= 4
        issue(0, 0)

        def chunkf(j, carry):
            @pl.when((j & 1) == 0)
            def _():
                step(j, 0)

            @pl.when((j & 1) == 1)
            def _():
                step(j, 1)

            return carry

        lax.fori_loop(0, ch, chunkf, 0)
        sc_wait(ch - 1, (ch - 1) & 1)
        plsc.subcore_barrier()

        pltpu.sync_copy(acc.at[pl.ds(base, rps)],
                        acc_h.at[c, pl.ds(base, rps)])
        if with_deg:
            pltpu.sync_copy(degtile, deg_h.at[wid])

    return pl.kernel(body, out_type=out_type, mesh=mesh, scratch_types=scratch,
                     name=f"sc_spmm_d{d}_ch{ch}_n{n_out}_w{int(weighted)}"
                          f"_g{int(with_deg)}")


def _make_sc_gather(d, ch):
    mesh = plsc.VectorSubcoreMesh(core_axis_name="c", subcore_axis_name="s")
    out_type = jax.ShapeDtypeStruct((NWORK * ch * CK, d), jnp.float32)
    scratch = [
        pltpu.VMEM((ch * CK,), jnp.int32),
        pltpu.VMEM((CK, d), jnp.float32),
        pltpu.SemaphoreType.DMA,
    ]

    def body(y_h, idx_h, out_h, idx_v, rows_v, sem):
        c = lax.axis_index("c")
        s = lax.axis_index("s")
        wid = s * SC_CORES + c
        pltpu.sync_copy(idx_h.at[pl.ds(wid * ch * CK, ch * CK)], idx_v)

        def chunk(jc, carry):
            pltpu.async_copy(y_h.at[idx_v.at[pl.ds(jc * CK, CK)]],
                             rows_v, sem).wait()
            pltpu.sync_copy(rows_v, out_h.at[pl.ds((wid * ch + jc) * CK, CK)])
            return carry

        lax.fori_loop(0, ch, chunk, 0)

    return pl.kernel(body, out_type=out_type, mesh=mesh, scratch_types=scratch,
                     name=f"sc_gather_d{d}_ch{ch}")


def _pad_edges(src, dst, w, n_in, n_out):
    e = src.shape[0]
    ch = _cdiv(_cdiv(e, NWORK * CK), 8) * 8
    epad = NWORK * CK * ch
    n_acc = _cdiv(n_out + 1, SC_SUBCORES * 8) * SC_SUBCORES * 8
    pad_ids = jnp.arange(epad - e, dtype=jnp.int32)
    if w is not None:
        pad_dst = pad_ids % n_out
        pw = jnp.concatenate(
            [w, jnp.zeros((epad - e,), jnp.float32)]).reshape(-1, CK)
    else:
        pad_dst = n_out + pad_ids % (n_acc - n_out)
        pw = None
    psrc = jnp.concatenate([src, pad_ids % n_in]).reshape(-1, CK)
    pdst = jnp.concatenate([dst, pad_dst]).reshape(-1, CK)
    return psrc, pdst, pw, ch


def _lin1_body(x_ref, w_ref, o_ref):
    o_ref[...] = jnp.dot(x_ref[...], w_ref[...],
                         preferred_element_type=jnp.float32)


def _relu_body(p0_ref, p1_ref, p2_ref, p3_ref, b_ref, o_ref):
    o_ref[...] = jnp.maximum(
        p0_ref[...] + p1_ref[...] + p2_ref[...] + p3_ref[...] + b_ref[...],
        0.0)


def _mid_body(x0_ref, x1_ref, wp_ref, wu_ref, nw_ref, tab_ref,
              psi_ref, base_ref):
    xc = x0_ref[...] + x1_ref[...]
    psi_ref[...] = jnp.dot(xc, wp_ref[...], preferred_element_type=jnp.float32)
    iot = lax.broadcasted_iota(jnp.int32, (N1, MAXW), 1)
    oh = (nw_ref[...] == iot).astype(jnp.float32)
    base_ref[...] = (
        jnp.dot(xc, wu_ref[...], preferred_element_type=jnp.float32)
        + jnp.dot(oh, tab_ref[...], preferred_element_type=jnp.float32))


def _xc2_body(base_ref, m0_ref, m1_ref, d_ref, o_ref):
    o_ref[...] = jnp.maximum(
        base_ref[...] + (m0_ref[...] + m1_ref[...]) / (d_ref[...] + 1.0), 0.0)


def _ref_body(g_ref, h_ref, w2_ref, refo_ref, z_ref):
    r = g_ref[...] + h_ref[...]
    refo_ref[...] = r
    z_ref[...] = jnp.dot(r, w2_ref[...], preferred_element_type=jnp.float32)


def _out_body(p0_ref, p1_ref, p2_ref, p3_ref, b_ref, o_ref):
    o_ref[...] = p0_ref[...] + p1_ref[...] + p2_ref[...] + p3_ref[...] + b_ref[...]


def _sds(shape):
    return jax.ShapeDtypeStruct(shape, jnp.float32)


def kernel(x, A0_idx, A0_w, A1_idx, A1_w, assign, node_wgt,
           W1, b1, W2, b2, Wp, Wu, wgt_table):
    src0 = A0_idx[0].astype(jnp.int32)
    dst0 = A0_idx[1].astype(jnp.int32)
    src1 = A1_idx[0].astype(jnp.int32)
    dst1 = A1_idx[1].astype(jnp.int32)
    assign_i = assign.astype(jnp.int32)

    y = pl.pallas_call(_lin1_body, out_shape=_sds((N0, NH)))(x, W1)

    s0, d0, w0, ch0 = _pad_edges(src0, dst0, A0_w.astype(jnp.float32), N0, N0)
    chh = ch0 // 2
    hrows = NWORK * chh
    spmm_half = _make_sc_spmm(NH, chh, N0, weighted=True, with_deg=False)
    hpa = spmm_half(y, s0[:hrows], d0[:hrows], w0[:hrows])[0]
    hpb = spmm_half(y, s0[hrows:], d0[hrows:], w0[hrows:])[0]
    h = pl.pallas_call(_relu_body, out_shape=_sds((N0, NH)))(
        hpa[0, :N0], hpa[1, :N0], hpb[0, :N0], hpb[1, :N0],
        b1.reshape(1, NH))

    siota = jnp.arange(N0, dtype=jnp.int32)
    ps, pd, _, chp = _pad_edges(siota, assign_i, None, N0, N1)
    xcp = _make_sc_spmm(NH, chp, N1, weighted=False, with_deg=False)(
        h, ps, pd)[0]

    nwb = jnp.broadcast_to(node_wgt.astype(jnp.int32)[:, None], (N1, MAXW))
    psi, basev = pl.pallas_call(
        _mid_body, out_shape=[_sds((N1, NH)), _sds((N1, NH))])(
        xcp[0, :N1], xcp[1, :N1], Wp, Wu, nwb, wgt_table)

    s1, d1, w1, ch1 = _pad_edges(src1, dst1, A1_w.astype(jnp.float32), N1, N1)
    mp, dp = _make_sc_spmm(NH, ch1, N1, weighted=True, with_deg=True)(
        psi, s1, d1, w1)
    dsum = jnp.sum(dp, axis=0).reshape(-1)[:N1].reshape(N1, 1)
    xc2 = pl.pallas_call(_xc2_body, out_shape=_sds((N1, NH)))(
        basev, mp[0, :N1], mp[1, :N1], dsum)

    chg = _cdiv(N0, NWORK * CK)
    npadg = NWORK * CK * chg
    apad = jnp.concatenate(
        [assign_i,
         jnp.arange(npadg - N0, dtype=jnp.int32) % N1])
    g = _make_sc_gather(NH, chg)(xc2, apad)
    w2p = jnp.pad(W2, ((0, 0), (0, 128 - NCLS)))
    ref_h, z = pl.pallas_call(
        _ref_body, out_shape=[_sds((N0, NH)), _sds((N0, 128))])(
        g[:N0], h, w2p)

    opa = spmm_half(z, s0[:hrows], d0[:hrows], w0[:hrows])[0]
    opb = spmm_half(z, s0[hrows:], d0[hrows:], w0[hrows:])[0]
    b2p = jnp.pad(b2, (0, 128 - NCLS)).reshape(1, 128)
    out128 = pl.pallas_call(_out_body, out_shape=_sds((N0, 128)))(
        opa[0, :N0], opa[1, :N0], opb[0, :N0], opb[1, :N0], b2p)
    return (out128[:, :NCLS], h, ref_h)


# speedup vs baseline: 1.1647x; 1.1647x over previous
import jax
import jax.numpy as jnp
from jax import lax
from jax.experimental import pallas as pl
from jax.experimental.pallas import tpu as pltpu
from jax.experimental.pallas import tpu_sc as plsc

N0 = 10000
N1 = 5000
NF = 128
NH = 128
NCLS = 40
MAXW = 64

SC_CORES = 2
SC_SUBCORES = 16
NWORK = SC_CORES * SC_SUBCORES
CK = 128


def _cdiv(a, b):
    return (a + b - 1) // b


def _make_sc_spmm(d, ch, n_out, weighted, with_deg):
    n_acc = _cdiv(n_out + 1, SC_SUBCORES * 8) * SC_SUBCORES * 8
    rps = n_acc // SC_SUBCORES
    mesh = plsc.VectorSubcoreMesh(core_axis_name="c", subcore_axis_name="s")

    dr = n_acc // CK
    out_type = [jax.ShapeDtypeStruct((SC_CORES, n_acc, d), jnp.float32)]
    if with_deg:
        out_type.append(jax.ShapeDtypeStruct((NWORK, dr, CK), jnp.float32))

    scratch = [
        pltpu.VMEM((ch, CK), jnp.int32),
        pltpu.VMEM((ch, CK), jnp.int32),
    ]
    if weighted:
        scratch.append(pltpu.VMEM((ch, CK), jnp.float32))
    scratch.append(pltpu.VMEM((2, CK, d), jnp.float32))
    if with_deg:
        scratch.append(pltpu.VMEM((dr, CK), jnp.float32))
    scratch.append(pltpu.VMEM_SHARED((n_acc, d), jnp.float32))
    scratch.append(pltpu.SemaphoreType.DMA)
    scratch.append(pltpu.SemaphoreType.DMA)
    scratch.append(pltpu.SemaphoreType.DMA)
    scratch.append(pltpu.SemaphoreType.DMA)

    def body(*refs):
        it = iter(refs)
        y_h = next(it)
        src_h = next(it)
        dst_h = next(it)
        w_h = next(it) if weighted else None
        acc_h = next(it)
        deg_h = next(it) if with_deg else None
        src_v = next(it)
        dst_v = next(it)
        w_v = next(it) if weighted else None
        rows2_v = next(it)
        degtile = next(it) if with_deg else None
        acc = next(it)
        sems = (next(it), next(it))
        ssems = (next(it), next(it))
        bufs = (rows2_v.at[0], rows2_v.at[1])

        c = lax.axis_index("c")
        s = lax.axis_index("s")
        wid = s * SC_CORES + c

        def zrow(e, carry):
            z = jnp.zeros((16,), jnp.float32)
            for j in range(d // 16):
                rows2_v[0, e, pl.ds(j * 16, 16)] = z
            return carry

        lax.fori_loop(0, CK, zrow, 0)
        if with_deg:
            def zdeg(r, carry):
                z = jnp.zeros((16,), jnp.float32)
                for j in range(CK // 16):
                    degtile[r, pl.ds(j * 16, 16)] = z
                return carry

            lax.fori_loop(0, dr, zdeg, 0)
        base = s * rps
        off = 0
        while off < rps:
            csz = min(rps - off, CK)
            pltpu.sync_copy(bufs[0].at[pl.ds(0, csz)],
                            acc.at[pl.ds(base + off, csz)])
            off += csz
        plsc.subcore_barrier()

        pltpu.sync_copy(src_h.at[pl.ds(wid * ch, ch)], src_v)
        pltpu.sync_copy(dst_h.at[pl.ds(wid * ch, ch)], dst_v)
        if weighted:
            pltpu.sync_copy(w_h.at[pl.ds(wid * ch, ch)], w_v)

        def issue(jc, b):
            return pltpu.async_copy(y_h.at[src_v.at[jc]], bufs[b], sems[b])

        def wait(jc, b):
            pltpu.make_async_copy(y_h.at[src_v.at[jc]], bufs[b],
                                  sems[b]).wait()

        def process(jc, b):
            buf = bufs[b]
            wait(jc, b)
            if weighted:
                def scale(g, cc):
                    wv = w_v[jc, pl.ds(g * 16, 16)]
                    dv = dst_v[jc, pl.ds(g * 16, 16)] if with_deg else None
                    for e16 in range(16):
                        we = wv[e16]
                        e = g * 16 + e16
                        for j in range(d // 16):
                            sl = pl.ds(j * 16, 16)
                            buf[e, sl] = buf[e, sl] * we
                        if with_deg:
                            de = dv[e16]
                            r = de >> 7
                            cb = ((de >> 4) & 7) * 16
                            lane = de & 15
                            lanes = lax.iota(jnp.int32, 16)
                            cur = degtile[r, pl.ds(cb, 16)]
                            degtile[r, pl.ds(cb, 16)] = cur + jnp.where(
                                lanes == lane, we, 0.0)
                    return cc

                lax.fori_loop(0, CK // 16, scale, 0)

        def sc_start(jc, b):
            pltpu.async_copy(bufs[b], acc.at[dst_v.at[jc]], ssems[b],
                             add=True)

        def sc_wait(jc, b):
            pltpu.make_async_copy(bufs[b], acc.at[dst_v.at[jc]],
                                  ssems[b]).wait()

        assert ch % 2 == 0 and ch >= 4
        issue(0, 0)
        issue(1, 1)

        def chunk2(i, carry):
            j0 = 2 * i
            process(j0, 0)
            sc_start(j0, 0)
            process(j0 + 1, 1)
            sc_wait(j0, 0)
            issue(j0 + 2, 0)
            pltpu.sync_copy(bufs[1], acc.at[dst_v.at[j0 + 1]], add=True)
            issue(j0 + 3, 1)
            return carry

        lax.fori_loop(0, ch // 2 - 1, chunk2, 0)
        process(ch - 2, 0)
        sc_start(ch - 2, 0)
        process(ch - 1, 1)
        sc_wait(ch - 2, 0)
        pltpu.sync_copy(bufs[1], acc.at[dst_v.at[ch - 1]], add=True)
        plsc.subcore_barrier()

        pltpu.sync_copy(acc.at[pl.ds(base, rps)],
                        acc_h.at[c, pl.ds(base, rps)])
        if with_deg:
            pltpu.sync_copy(degtile, deg_h.at[wid])

    return pl.kernel(body, out_type=out_type, mesh=mesh, scratch_types=scratch,
                     name=f"sc_spmm_d{d}_ch{ch}_n{n_out}_w{int(weighted)}"
                          f"_g{int(with_deg)}")


def _make_sc_gather(d, ch):
    mesh = plsc.VectorSubcoreMesh(core_axis_name="c", subcore_axis_name="s")
    out_type = jax.ShapeDtypeStruct((NWORK * ch * CK, d), jnp.float32)
    scratch = [
        pltpu.VMEM((ch * CK,), jnp.int32),
        pltpu.VMEM((CK, d), jnp.float32),
        pltpu.SemaphoreType.DMA,
    ]

    def body(y_h, idx_h, out_h, idx_v, rows_v, sem):
        c = lax.axis_index("c")
        s = lax.axis_index("s")
        wid = s * SC_CORES + c
        pltpu.sync_copy(idx_h.at[pl.ds(wid * ch * CK, ch * CK)], idx_v)

        def chunk(jc, carry):
            pltpu.async_copy(y_h.at[idx_v.at[pl.ds(jc * CK, CK)]],
                             rows_v, sem).wait()
            pltpu.sync_copy(rows_v, out_h.at[pl.ds((wid * ch + jc) * CK, CK)])
            return carry

        lax.fori_loop(0, ch, chunk, 0)

    return pl.kernel(body, out_type=out_type, mesh=mesh, scratch_types=scratch,
                     name=f"sc_gather_d{d}_ch{ch}")


def _pad_edges(src, dst, w, n_in, n_out):
    e = src.shape[0]
    ch = _cdiv(_cdiv(e, NWORK * CK), 8) * 8
    epad = NWORK * CK * ch
    n_acc = _cdiv(n_out + 1, SC_SUBCORES * 8) * SC_SUBCORES * 8
    pad_ids = jnp.arange(epad - e, dtype=jnp.int32)
    if w is not None:
        pad_dst = pad_ids % n_out
        pw = jnp.concatenate(
            [w, jnp.zeros((epad - e,), jnp.float32)]).reshape(-1, CK)
    else:
        pad_dst = n_out + pad_ids % (n_acc - n_out)
        pw = None
    psrc = jnp.concatenate([src, pad_ids % n_in]).reshape(-1, CK)
    pdst = jnp.concatenate([dst, pad_dst]).reshape(-1, CK)
    return psrc, pdst, pw, ch


def _lin1_body(x_ref, w_ref, o_ref):
    o_ref[...] = jnp.dot(x_ref[...], w_ref[...],
                         preferred_element_type=jnp.float32)


def _relu_body(p0_ref, p1_ref, p2_ref, p3_ref, b_ref, o_ref):
    o_ref[...] = jnp.maximum(
        p0_ref[...] + p1_ref[...] + p2_ref[...] + p3_ref[...] + b_ref[...],
        0.0)


def _mid_body(x0_ref, x1_ref, wp_ref, wu_ref, nw_ref, tab_ref,
              psi_ref, base_ref):
    xc = x0_ref[...] + x1_ref[...]
    psi_ref[...] = jnp.dot(xc, wp_ref[...], preferred_element_type=jnp.float32)
    iot = lax.broadcasted_iota(jnp.int32, (N1, MAXW), 1)
    oh = (nw_ref[...] == iot).astype(jnp.float32)
    base_ref[...] = (
        jnp.dot(xc, wu_ref[...], preferred_element_type=jnp.float32)
        + jnp.dot(oh, tab_ref[...], preferred_element_type=jnp.float32))


def _xc2_body(base_ref, m0_ref, m1_ref, d_ref, o_ref):
    o_ref[...] = jnp.maximum(
        base_ref[...] + (m0_ref[...] + m1_ref[...]) / (d_ref[...] + 1.0), 0.0)


def _ref_body(g_ref, h_ref, w2_ref, refo_ref, z_ref):
    r = g_ref[...] + h_ref[...]
    refo_ref[...] = r
    z_ref[...] = jnp.dot(r, w2_ref[...], preferred_element_type=jnp.float32)


def _out_body(p0_ref, p1_ref, p2_ref, p3_ref, b_ref, o_ref):
    o_ref[...] = p0_ref[...] + p1_ref[...] + p2_ref[...] + p3_ref[...] + b_ref[...]


def _sds(shape):
    return jax.ShapeDtypeStruct(shape, jnp.float32)


def kernel(x, A0_idx, A0_w, A1_idx, A1_w, assign, node_wgt,
           W1, b1, W2, b2, Wp, Wu, wgt_table):
    src0 = A0_idx[0].astype(jnp.int32)
    dst0 = A0_idx[1].astype(jnp.int32)
    src1 = A1_idx[0].astype(jnp.int32)
    dst1 = A1_idx[1].astype(jnp.int32)
    assign_i = assign.astype(jnp.int32)

    y = pl.pallas_call(_lin1_body, out_shape=_sds((N0, NH)))(x, W1)

    s0, d0, w0, ch0 = _pad_edges(src0, dst0, A0_w.astype(jnp.float32), N0, N0)
    chh = ch0 // 2
    hrows = NWORK * chh
    spmm_half = _make_sc_spmm(NH, chh, N0, weighted=True, with_deg=False)
    hpa = spmm_half(y, s0[:hrows], d0[:hrows], w0[:hrows])[0]
    hpb = spmm_half(y, s0[hrows:], d0[hrows:], w0[hrows:])[0]
    h = pl.pallas_call(_relu_body, out_shape=_sds((N0, NH)))(
        hpa[0, :N0], hpa[1, :N0], hpb[0, :N0], hpb[1, :N0],
        b1.reshape(1, NH))

    siota = jnp.arange(N0, dtype=jnp.int32)
    ps, pd, _, chp = _pad_edges(siota, assign_i, None, N0, N1)
    xcp = _make_sc_spmm(NH, chp, N1, weighted=False, with_deg=False)(
        h, ps, pd)[0]

    nwb = jnp.broadcast_to(node_wgt.astype(jnp.int32)[:, None], (N1, MAXW))
    psi, basev = pl.pallas_call(
        _mid_body, out_shape=[_sds((N1, NH)), _sds((N1, NH))])(
        xcp[0, :N1], xcp[1, :N1], Wp, Wu, nwb, wgt_table)

    s1, d1, w1, ch1 = _pad_edges(src1, dst1, A1_w.astype(jnp.float32), N1, N1)
    mp, dp = _make_sc_spmm(NH, ch1, N1, weighted=True, with_deg=True)(
        psi, s1, d1, w1)
    dsum = jnp.sum(dp, axis=0).reshape(-1)[:N1].reshape(N1, 1)
    xc2 = pl.pallas_call(_xc2_body, out_shape=_sds((N1, NH)))(
        basev, mp[0, :N1], mp[1, :N1], dsum)

    chg = _cdiv(N0, NWORK * CK)
    npadg = NWORK * CK * chg
    apad = jnp.concatenate(
        [assign_i,
         jnp.arange(npadg - N0, dtype=jnp.int32) % N1])
    g = _make_sc_gather(NH, chg)(xc2, apad)
    w2p = jnp.pad(W2, ((0, 0), (0, 128 - NCLS)))
    ref_h, z = pl.pallas_call(
        _ref_body, out_shape=[_sds((N0, NH)), _sds((N0, 128))])(
        g[:N0], h, w2p)

    opa = spmm_half(z, s0[:hrows], d0[:hrows], w0[:hrows])[0]
    opb = spmm_half(z, s0[hrows:], d0[hrows:], w0[hrows:])[0]
    b2p = jnp.pad(b2, (0, 128 - NCLS)).reshape(1, 128)
    out128 = pl.pallas_call(_out_body, out_shape=_sds((N0, 128)))(
        opa[0, :N0], opa[1, :N0], opb[0, :N0], opb[1, :N0], b2p)
    return (out128[:, :NCLS], h, ref_h)


# speedup vs baseline: 1.2021x; 1.0320x over previous
import jax
import jax.numpy as jnp
from jax import lax
from jax.experimental import pallas as pl
from jax.experimental.pallas import tpu as pltpu
from jax.experimental.pallas import tpu_sc as plsc

N0 = 10000
N1 = 5000
NF = 128
NH = 128
NCLS = 40
MAXW = 64

SC_CORES = 2
SC_SUBCORES = 16
NWORK = SC_CORES * SC_SUBCORES
CK = 128


def _cdiv(a, b):
    return (a + b - 1) // b


def _make_sc_spmm(d, ch, n_out, weighted, with_deg):
    n_acc = _cdiv(n_out + 1, SC_SUBCORES * 8) * SC_SUBCORES * 8
    rps = n_acc // SC_SUBCORES
    mesh = plsc.VectorSubcoreMesh(core_axis_name="c", subcore_axis_name="s")

    dr = n_acc // CK
    out_type = [jax.ShapeDtypeStruct((SC_CORES, n_acc, d), jnp.float32)]
    if with_deg:
        out_type.append(jax.ShapeDtypeStruct((NWORK, dr, CK), jnp.float32))

    scratch = [
        pltpu.VMEM((ch, CK), jnp.int32),
        pltpu.VMEM((ch, CK), jnp.int32),
    ]
    if weighted:
        scratch.append(pltpu.VMEM((ch, CK), jnp.float32))
    scratch.append(pltpu.VMEM((2, CK, d), jnp.float32))
    if with_deg:
        scratch.append(pltpu.VMEM((dr, CK), jnp.float32))
    scratch.append(pltpu.VMEM_SHARED((n_acc, d), jnp.float32))
    scratch.append(pltpu.SemaphoreType.DMA)
    scratch.append(pltpu.SemaphoreType.DMA)
    scratch.append(pltpu.SemaphoreType.DMA)
    scratch.append(pltpu.SemaphoreType.DMA)

    def body(*refs):
        it = iter(refs)
        y_h = next(it)
        src_h = next(it)
        dst_h = next(it)
        w_h = next(it) if weighted else None
        acc_h = next(it)
        deg_h = next(it) if with_deg else None
        src_v = next(it)
        dst_v = next(it)
        w_v = next(it) if weighted else None
        rows2_v = next(it)
        degtile = next(it) if with_deg else None
        acc = next(it)
        sems = (next(it), next(it))
        ssems = (next(it), next(it))
        bufs = (rows2_v.at[0], rows2_v.at[1])

        c = lax.axis_index("c")
        s = lax.axis_index("s")
        wid = s * SC_CORES + c

        def zrow(e, carry):
            z = jnp.zeros((16,), jnp.float32)
            for j in range(d // 16):
                rows2_v[0, e, pl.ds(j * 16, 16)] = z
            return carry

        lax.fori_loop(0, CK, zrow, 0)
        if with_deg:
            def zdeg(r, carry):
                z = jnp.zeros((16,), jnp.float32)
                for j in range(CK // 16):
                    degtile[r, pl.ds(j * 16, 16)] = z
                return carry

            lax.fori_loop(0, dr, zdeg, 0)
        base = s * rps
        off = 0
        while off < rps:
            csz = min(rps - off, CK)
            pltpu.sync_copy(bufs[0].at[pl.ds(0, csz)],
                            acc.at[pl.ds(base + off, csz)])
            off += csz
        plsc.subcore_barrier()

        pltpu.sync_copy(src_h.at[pl.ds(wid * ch, ch)], src_v)
        pltpu.sync_copy(dst_h.at[pl.ds(wid * ch, ch)], dst_v)
        if weighted:
            pltpu.sync_copy(w_h.at[pl.ds(wid * ch, ch)], w_v)

        def issue(jc, b):
            return pltpu.async_copy(y_h.at[src_v.at[jc]], bufs[b], sems[b])

        def wait(jc, b):
            pltpu.make_async_copy(y_h.at[src_v.at[jc]], bufs[b],
                                  sems[b]).wait()

        def process(jc, b):
            buf = bufs[b]
            wait(jc, b)
            if weighted:
                def scale(g, cc):
                    wv = w_v[jc, pl.ds(g * 16, 16)]
                    dv = dst_v[jc, pl.ds(g * 16, 16)] if with_deg else None
                    for e16 in range(16):
                        we = wv[e16]
                        e = g * 16 + e16
                        for j in range(d // 16):
                            sl = pl.ds(j * 16, 16)
                            buf[e, sl] = buf[e, sl] * we
                        if with_deg:
                            de = dv[e16]
                            r = de >> 7
                            cb = ((de >> 4) & 7) * 16
                            lane = de & 15
                            lanes = lax.iota(jnp.int32, 16)
                            cur = degtile[r, pl.ds(cb, 16)]
                            degtile[r, pl.ds(cb, 16)] = cur + jnp.where(
                                lanes == lane, we, 0.0)
                    return cc

                lax.fori_loop(0, CK // 16, scale, 0)

        def sc_start(jc, b):
            pltpu.async_copy(bufs[b], acc.at[dst_v.at[jc]], ssems[b],
                             add=True)

        def sc_wait(jc, b):
            pltpu.make_async_copy(bufs[b], acc.at[dst_v.at[jc]],
                                  ssems[b]).wait()

        def scatter(jc, b):
            pltpu.sync_copy(bufs[b], acc.at[dst_v.at[jc]], add=True)

        assert ch % 2 == 0 and ch >= 4
        issue(0, 0)
        issue(1, 1)

        def chunk2(i, carry):
            j0 = 2 * i
            process(j0, 0)
            scatter(j0, 0)
            issue(j0 + 2, 0)
            process(j0 + 1, 1)
            scatter(j0 + 1, 1)
            issue(j0 + 3, 1)
            return carry

        lax.fori_loop(0, ch // 2 - 1, chunk2, 0)
        process(ch - 2, 0)
        scatter(ch - 2, 0)
        process(ch - 1, 1)
        scatter(ch - 1, 1)
        plsc.subcore_barrier()

        pltpu.sync_copy(acc.at[pl.ds(base, rps)],
                        acc_h.at[c, pl.ds(base, rps)])
        if with_deg:
            pltpu.sync_copy(degtile, deg_h.at[wid])

    return pl.kernel(body, out_type=out_type, mesh=mesh, scratch_types=scratch,
                     name=f"sc_spmm_d{d}_ch{ch}_n{n_out}_w{int(weighted)}"
                          f"_g{int(with_deg)}")


def _make_sc_gather(d, ch):
    mesh = plsc.VectorSubcoreMesh(core_axis_name="c", subcore_axis_name="s")
    out_type = jax.ShapeDtypeStruct((NWORK * ch * CK, d), jnp.float32)
    scratch = [
        pltpu.VMEM((ch * CK,), jnp.int32),
        pltpu.VMEM((CK, d), jnp.float32),
        pltpu.SemaphoreType.DMA,
    ]

    def body(y_h, idx_h, out_h, idx_v, rows_v, sem):
        c = lax.axis_index("c")
        s = lax.axis_index("s")
        wid = s * SC_CORES + c
        pltpu.sync_copy(idx_h.at[pl.ds(wid * ch * CK, ch * CK)], idx_v)

        def chunk(jc, carry):
            pltpu.async_copy(y_h.at[idx_v.at[pl.ds(jc * CK, CK)]],
                             rows_v, sem).wait()
            pltpu.sync_copy(rows_v, out_h.at[pl.ds((wid * ch + jc) * CK, CK)])
            return carry

        lax.fori_loop(0, ch, chunk, 0)

    return pl.kernel(body, out_type=out_type, mesh=mesh, scratch_types=scratch,
                     name=f"sc_gather_d{d}_ch{ch}")


def _pad_edges(src, dst, w, n_in, n_out):
    e = src.shape[0]
    ch = _cdiv(_cdiv(e, NWORK * CK), 8) * 8
    epad = NWORK * CK * ch
    n_acc = _cdiv(n_out + 1, SC_SUBCORES * 8) * SC_SUBCORES * 8
    pad_ids = jnp.arange(epad - e, dtype=jnp.int32)
    if w is not None:
        pad_dst = pad_ids % n_out
        pw = jnp.concatenate(
            [w, jnp.zeros((epad - e,), jnp.float32)]).reshape(-1, CK)
    else:
        pad_dst = n_out + pad_ids % (n_acc - n_out)
        pw = None
    psrc = jnp.concatenate([src, pad_ids % n_in]).reshape(-1, CK)
    pdst = jnp.concatenate([dst, pad_dst]).reshape(-1, CK)
    return psrc, pdst, pw, ch


def _lin1_body(x_ref, w_ref, o_ref):
    o_ref[...] = jnp.dot(x_ref[...], w_ref[...],
                         preferred_element_type=jnp.float32)


def _relu_body(p0_ref, p1_ref, p2_ref, p3_ref, b_ref, o_ref):
    o_ref[...] = jnp.maximum(
        p0_ref[...] + p1_ref[...] + p2_ref[...] + p3_ref[...] + b_ref[...],
        0.0)


def _mid_body(x0_ref, x1_ref, wp_ref, wu_ref, nw_ref, tab_ref,
              psi_ref, base_ref):
    xc = x0_ref[...] + x1_ref[...]
    psi_ref[...] = jnp.dot(xc, wp_ref[...], preferred_element_type=jnp.float32)
    iot = lax.broadcasted_iota(jnp.int32, (N1, MAXW), 1)
    oh = (nw_ref[...] == iot).astype(jnp.float32)
    base_ref[...] = (
        jnp.dot(xc, wu_ref[...], preferred_element_type=jnp.float32)
        + jnp.dot(oh, tab_ref[...], preferred_element_type=jnp.float32))


def _xc2_body(base_ref, m0_ref, m1_ref, d_ref, o_ref):
    o_ref[...] = jnp.maximum(
        base_ref[...] + (m0_ref[...] + m1_ref[...]) / (d_ref[...] + 1.0), 0.0)


def _ref_body(g_ref, h_ref, w2_ref, refo_ref, z_ref):
    r = g_ref[...] + h_ref[...]
    refo_ref[...] = r
    z_ref[...] = jnp.dot(r, w2_ref[...], preferred_element_type=jnp.float32)


def _out_body(p0_ref, p1_ref, p2_ref, p3_ref, b_ref, o_ref):
    o_ref[...] = p0_ref[...] + p1_ref[...] + p2_ref[...] + p3_ref[...] + b_ref[...]


def _sds(shape):
    return jax.ShapeDtypeStruct(shape, jnp.float32)


def kernel(x, A0_idx, A0_w, A1_idx, A1_w, assign, node_wgt,
           W1, b1, W2, b2, Wp, Wu, wgt_table):
    src0 = A0_idx[0].astype(jnp.int32)
    dst0 = A0_idx[1].astype(jnp.int32)
    src1 = A1_idx[0].astype(jnp.int32)
    dst1 = A1_idx[1].astype(jnp.int32)
    assign_i = assign.astype(jnp.int32)

    y = pl.pallas_call(_lin1_body, out_shape=_sds((N0, NH)))(x, W1)

    s0, d0, w0, ch0 = _pad_edges(src0, dst0, A0_w.astype(jnp.float32), N0, N0)
    chh = ch0 // 2
    hrows = NWORK * chh
    spmm_half = _make_sc_spmm(NH, chh, N0, weighted=True, with_deg=False)
    hpa = spmm_half(y, s0[:hrows], d0[:hrows], w0[:hrows])[0]
    hpb = spmm_half(y, s0[hrows:], d0[hrows:], w0[hrows:])[0]
    h = pl.pallas_call(_relu_body, out_shape=_sds((N0, NH)))(
        hpa[0, :N0], hpa[1, :N0], hpb[0, :N0], hpb[1, :N0],
        b1.reshape(1, NH))

    siota = jnp.arange(N0, dtype=jnp.int32)
    ps, pd, _, chp = _pad_edges(siota, assign_i, None, N0, N1)
    xcp = _make_sc_spmm(NH, chp, N1, weighted=False, with_deg=False)(
        h, ps, pd)[0]

    nwb = jnp.broadcast_to(node_wgt.astype(jnp.int32)[:, None], (N1, MAXW))
    psi, basev = pl.pallas_call(
        _mid_body, out_shape=[_sds((N1, NH)), _sds((N1, NH))])(
        xcp[0, :N1], xcp[1, :N1], Wp, Wu, nwb, wgt_table)

    s1, d1, w1, ch1 = _pad_edges(src1, dst1, A1_w.astype(jnp.float32), N1, N1)
    mp, dp = _make_sc_spmm(NH, ch1, N1, weighted=True, with_deg=True)(
        psi, s1, d1, w1)
    dsum = jnp.sum(dp, axis=0).reshape(-1)[:N1].reshape(N1, 1)
    xc2 = pl.pallas_call(_xc2_body, out_shape=_sds((N1, NH)))(
        basev, mp[0, :N1], mp[1, :N1], dsum)

    chg = _cdiv(N0, NWORK * CK)
    npadg = NWORK * CK * chg
    apad = jnp.concatenate(
        [assign_i,
         jnp.arange(npadg - N0, dtype=jnp.int32) % N1])
    g = _make_sc_gather(NH, chg)(xc2, apad)
    w2p = jnp.pad(W2, ((0, 0), (0, 128 - NCLS)))
    ref_h, z = pl.pallas_call(
        _ref_body, out_shape=[_sds((N0, NH)), _sds((N0, 128))])(
        g[:N0], h, w2p)

    opa = spmm_half(z, s0[:hrows], d0[:hrows], w0[:hrows])[0]
    opb = spmm_half(z, s0[hrows:], d0[hrows:], w0[hrows:])[0]
    b2p = jnp.pad(b2, (0, 128 - NCLS)).reshape(1, 128)
    out128 = pl.pallas_call(_out_body, out_shape=_sds((N0, 128)))(
        opa[0, :N0], opa[1, :N0], opb[0, :N0], opb[1, :N0], b2p)
    return (out128[:, :NCLS], h, ref_h)
